# baseline (speedup 1.0000x reference)
import functools

import jax
import jax.numpy as jnp
from jax import lax
from jax.experimental import pallas as pl
from jax.experimental.pallas import tpu as pltpu
from jax.experimental.pallas import tpu_sc as plsc

N = 10000
E = 160000
D = 256
SCALE = 0.8
NC = 2
NS = 16
JUNK = N

H = 128

A_BATCHES = 40
A_SLOTS = A_BATCHES * 128
A_TOTAL = NC * NS * A_SLOTS
DEG_LEN = NC * NS * 640

EPT = E // NS
C_B = 128
C_NB = 79
C_PAD = C_NB * C_B - EPT
PASS_ROWS = 5000
ACC_ROWS = PASS_ROWS + 8
RPT = 312
TAIL_R0 = NS * RPT
TAIL = PASS_ROWS - TAIL_R0

_mesh = plsc.VectorSubcoreMesh(core_axis_name="c", subcore_axis_name="s")


@functools.partial(
    pl.kernel,
    mesh=_mesh,
    out_type=[jax.ShapeDtypeStruct((N,), jnp.float32),
              jax.ShapeDtypeStruct((N,), jnp.float32)],
    scratch_types=[
        pltpu.VMEM((A_BATCHES, 128), jnp.int32),
        pltpu.VMEM((128,), jnp.float32),
        pltpu.VMEM((640,), jnp.float32),
        pltpu.VMEM((N,), jnp.float32),
        pltpu.VMEM_SHARED((DEG_LEN,), jnp.float32),
    ],
)
def _deg_kernel(dstA, counts0, counts1, idx_v, ones_v, zb_v, cnt_v, deg):
    c = lax.axis_index("c")
    t = lax.axis_index("s")
    wid = t * NC + c
    pltpu.sync_copy(dstA.at[wid], idx_v)
    for i in range(8):
        ones_v[pl.ds(i * 16, 16)] = jnp.full((16,), 1.0, jnp.float32)
    for i in range(40):
        zb_v[pl.ds(i * 16, 16)] = jnp.zeros((16,), jnp.float32)
    pltpu.sync_copy(zb_v, deg.at[pl.ds(t * 640, 640)])
    plsc.subcore_barrier()

    def body(j, carry):
        pltpu.sync_copy(ones_v, deg.at[idx_v.at[j]], add=True)
        return carry

    lax.fori_loop(0, A_BATCHES, body, 0)
    plsc.subcore_barrier()

    @pl.when((t == 0) & (c == 0))
    def _():
        pltpu.sync_copy(deg.at[pl.ds(0, N)], cnt_v)
        pltpu.sync_copy(cnt_v, counts0)

    @pl.when((t == 0) & (c == 1))
    def _():
        pltpu.sync_copy(deg.at[pl.ds(0, N)], cnt_v)
        pltpu.sync_copy(cnt_v, counts1)


def _lin_body(x_ref, w1_ref, b1_ref, w2_ref, b2_ref, deg_ref,
              zp1_ref, zp2_ref):
    xb = x_ref[...]
    dinv = lax.rsqrt(deg_ref[...])
    y1 = lax.dot_general(xb, w1_ref[...], (((1,), (1,)), ((), ())),
                         preferred_element_type=jnp.float32) + b1_ref[...]
    z1 = y1 * dinv
    zp1_ref[0] = z1[:, :H]
    zp1_ref[1] = z1[:, H:]
    h = lax.dot_general(xb, w2_ref[...], (((1,), (1,)), ((), ())),
                        preferred_element_type=jnp.float32) + b2_ref[...]
    nrm = jnp.sqrt(jnp.sum(h * h, axis=1, keepdims=True))
    z2 = h * (SCALE / jnp.maximum(nrm, 1e-12)) * dinv
    zp2_ref[0] = z2[:, :H]
    zp2_ref[1] = z2[:, H:]


_lin = pl.pallas_call(
    _lin_body,
    grid=(25,),
    in_specs=[
        pl.BlockSpec((400, D), lambda i: (i, 0)),
        pl.BlockSpec((D, D), lambda i: (0, 0)),
        pl.BlockSpec((1, D), lambda i: (0, 0)),
        pl.BlockSpec((D, D), lambda i: (0, 0)),
        pl.BlockSpec((1, D), lambda i: (0, 0)),
        pl.BlockSpec((400, 1), lambda i: (i, 0)),
    ],
    out_specs=[pl.BlockSpec((NC, 400, H), lambda i: (0, i, 0))] * 2,
    out_shape=[jax.ShapeDtypeStruct((NC, N, H), jnp.float32)] * 2,
)


@functools.partial(
    pl.kernel,
    mesh=_mesh,
    out_type=[jax.ShapeDtypeStruct((NC, N, H), jnp.float32),
              jax.ShapeDtypeStruct((NC, N, H), jnp.float32)],
    scratch_types=[
        pltpu.VMEM((C_NB, C_B), jnp.int32),
        pltpu.VMEM((C_NB, C_B), jnp.int32),
        pltpu.VMEM((C_NB, C_B), jnp.int32),
        pltpu.VMEM((C_NB, C_B), jnp.int32),
        pltpu.VMEM((2, C_B, H), jnp.float32),
        pltpu.VMEM_SHARED((ACC_ROWS, H), jnp.float32),
        pltpu.SemaphoreType.DMA,
        pltpu.SemaphoreType.DMA,
    ],
)
def _prop_kernel(srcP, dstP, z1S, z2S, s1_out, s2_out,
                 src_v, dst_v, rem0, rem1, gbuf, acc, sem0, sem1):
    c = lax.axis_index("c")
    t = lax.axis_index("s")
    pltpu.sync_copy(srcP.at[c, t], src_v)
    pltpu.sync_copy(dstP.at[t], dst_v)
    sems = (sem0, sem1)

    def rm_body(i, carry):
        for k in range(C_B // 16):
            d = dst_v[i, pl.ds(16 * k, 16)]
            rem0[i, pl.ds(16 * k, 16)] = jnp.where(
                d < PASS_ROWS, d, PASS_ROWS)
            rem1[i, pl.ds(16 * k, 16)] = jnp.where(
                d >= PASS_ROWS, d - PASS_ROWS, PASS_ROWS)
        return carry

    lax.fori_loop(0, C_NB, rm_body, 0)

    def sweep(zS, s_out, base, rem):
        def g_start(j, b):
            pltpu.make_async_copy(zS.at[src_v.at[j]], gbuf.at[b],
                                  sems[b]).start()

        def g_wait(j, b):
            pltpu.make_async_copy(zS.at[src_v.at[j]], gbuf.at[b],
                                  sems[b]).wait()

        g_start(0, 0)
        g_start(1, 1)
        pltpu.sync_copy(zS.at[pl.ds(c * N + base + t * RPT, RPT)],
                        acc.at[pl.ds(t * RPT, RPT)])

        @pl.when(t == 0)
        def _():
            pltpu.sync_copy(zS.at[pl.ds(c * N + base + TAIL_R0, TAIL)],
                            acc.at[pl.ds(TAIL_R0, TAIL)])

        plsc.subcore_barrier()

        def body(i, carry):
            j0 = 2 * i
            for b in range(2):
                j = j0 + b

                @pl.when(j < C_NB)
                def _():
                    g_wait(j, b)
                    pltpu.sync_copy(gbuf.at[b], acc.at[rem.at[j]], add=True)

                    @pl.when(j + 2 < C_NB)
                    def _():
                        g_start(j + 2, b)

            return carry

        lax.fori_loop(0, (C_NB + 1) // 2, body, 0)
        plsc.subcore_barrier()
        pltpu.sync_copy(acc.at[pl.ds(t * RPT, RPT)],
                        s_out.at[c, pl.ds(base + t * RPT, RPT)])

        @pl.when(t == 0)
        def _():
            pltpu.sync_copy(acc.at[pl.ds(TAIL_R0, TAIL)],
                            s_out.at[c, pl.ds(base + TAIL_R0, TAIL)])

    sweep(z1S, s1_out, 0, rem0)
    sweep(z1S, s1_out, PASS_ROWS, rem1)
    sweep(z2S, s2_out, 0, rem0)
    sweep(z2S, s2_out, PASS_ROWS, rem1)


def _out_body(s1_ref, s2_ref, deg_ref, h_ref, x_ref):
    dinv = lax.rsqrt(deg_ref[...])
    s1 = s1_ref[...]
    s2 = s2_ref[...]
    x_ref[...] = jnp.concatenate([s1[0], s1[1]], axis=1) * dinv
    h_ref[...] = jnp.concatenate([s2[0], s2[1]], axis=1) * dinv


_out = pl.pallas_call(
    _out_body,
    grid=(25,),
    in_specs=[
        pl.BlockSpec((NC, 400, H), lambda i: (0, i, 0)),
        pl.BlockSpec((NC, 400, H), lambda i: (0, i, 0)),
        pl.BlockSpec((400, 1), lambda i: (i, 0)),
    ],
    out_specs=[pl.BlockSpec((400, D), lambda i: (i, 0))] * 2,
    out_shape=[jax.ShapeDtypeStruct((N, D), jnp.float32)] * 2,
)


def kernel(x, edge_index, W1, b1, W2, b2):
    src = edge_index[0]
    dst = edge_index[1]
    padA = jnp.full((A_TOTAL - E,), JUNK, jnp.int32)
    dstA = jnp.concatenate([dst, padA]).reshape(NC * NS, A_BATCHES, 128)
    c0, c1 = _deg_kernel(dstA)
    deg_col = (c0 + c1 + 1.0).reshape(N, 1)
    zp1, zp2 = _lin(x, W1, b1.reshape(1, D), W2, b2.reshape(1, D), deg_col)
    srcG = jnp.concatenate(
        [src.reshape(NS, EPT), jnp.zeros((NS, C_PAD), jnp.int32)],
        axis=1).reshape(NS, C_NB, C_B)
    srcP = jnp.stack([srcG, srcG + N])
    dstP = jnp.concatenate(
        [dst.reshape(NS, EPT), jnp.full((NS, C_PAD), N, jnp.int32)],
        axis=1).reshape(NS, C_NB, C_B)
    s1, s2 = _prop_kernel(srcP, dstP, zp1.reshape(NC * N, H),
                          zp2.reshape(NC * N, H))
    h, x_ = _out(s1, s2, deg_col)
    return (h, x_)


# speedup vs baseline: 1.4373x; 1.4373x over previous
import functools

import jax
import jax.numpy as jnp
from jax import lax
from jax.experimental import pallas as pl
from jax.experimental.pallas import tpu as pltpu
from jax.experimental.pallas import tpu_sc as plsc

N = 10000
E = 160000
D = 256
SCALE = 0.8
NC = 2
NS = 16
JUNK = N

H = 128

A_BATCHES = 40
A_SLOTS = A_BATCHES * 128
A_TOTAL = NC * NS * A_SLOTS
DEG_LEN = NC * NS * 640

EPT = E // NS
C_B = 80
C_NB = EPT // C_B
PASS_ROWS = 5000
ACC_ROWS = PASS_ROWS + 8
RPT = 312
TAIL_R0 = NS * RPT
TAIL = PASS_ROWS - TAIL_R0

_mesh = plsc.VectorSubcoreMesh(core_axis_name="c", subcore_axis_name="s")


@functools.partial(
    pl.kernel,
    mesh=_mesh,
    out_type=[jax.ShapeDtypeStruct((N,), jnp.float32),
              jax.ShapeDtypeStruct((N,), jnp.float32)],
    scratch_types=[
        pltpu.VMEM((A_BATCHES, 128), jnp.int32),
        pltpu.VMEM((128,), jnp.float32),
        pltpu.VMEM((640,), jnp.float32),
        pltpu.VMEM((N,), jnp.float32),
        pltpu.VMEM_SHARED((DEG_LEN,), jnp.float32),
    ],
)
def _deg_kernel(dstA, counts0, counts1, idx_v, ones_v, zb_v, cnt_v, deg):
    c = lax.axis_index("c")
    t = lax.axis_index("s")
    wid = t * NC + c
    pltpu.sync_copy(dstA.at[wid], idx_v)
    for i in range(8):
        ones_v[pl.ds(i * 16, 16)] = jnp.full((16,), 1.0, jnp.float32)
    for i in range(40):
        zb_v[pl.ds(i * 16, 16)] = jnp.zeros((16,), jnp.float32)
    pltpu.sync_copy(zb_v, deg.at[pl.ds(t * 640, 640)])
    plsc.subcore_barrier()

    def body(j, carry):
        pltpu.sync_copy(ones_v, deg.at[idx_v.at[j]], add=True)
        return carry

    lax.fori_loop(0, A_BATCHES, body, 0)
    plsc.subcore_barrier()

    @pl.when((t == 0) & (c == 0))
    def _():
        pltpu.sync_copy(deg.at[pl.ds(0, N)], cnt_v)
        pltpu.sync_copy(cnt_v, counts0)

    @pl.when((t == 0) & (c == 1))
    def _():
        pltpu.sync_copy(deg.at[pl.ds(0, N)], cnt_v)
        pltpu.sync_copy(cnt_v, counts1)


def _lin_body(x_ref, w1_ref, b1_ref, w2_ref, b2_ref, deg_ref,
              zp1_ref, zp2_ref):
    xb = x_ref[...]
    dinv = lax.rsqrt(deg_ref[...])
    y1 = lax.dot_general(xb, w1_ref[...], (((1,), (1,)), ((), ())),
                         preferred_element_type=jnp.float32) + b1_ref[...]
    z1 = y1 * dinv
    zp1_ref[0] = z1[:, :H]
    zp1_ref[1] = z1[:, H:]
    h = lax.dot_general(xb, w2_ref[...], (((1,), (1,)), ((), ())),
                        preferred_element_type=jnp.float32) + b2_ref[...]
    nrm = jnp.sqrt(jnp.sum(h * h, axis=1, keepdims=True))
    z2 = h * (SCALE / jnp.maximum(nrm, 1e-12)) * dinv
    zp2_ref[0] = z2[:, :H]
    zp2_ref[1] = z2[:, H:]


_lin = pl.pallas_call(
    _lin_body,
    grid=(25,),
    in_specs=[
        pl.BlockSpec((400, D), lambda i: (i, 0)),
        pl.BlockSpec((D, D), lambda i: (0, 0)),
        pl.BlockSpec((1, D), lambda i: (0, 0)),
        pl.BlockSpec((D, D), lambda i: (0, 0)),
        pl.BlockSpec((1, D), lambda i: (0, 0)),
        pl.BlockSpec((400, 1), lambda i: (i, 0)),
    ],
    out_specs=[pl.BlockSpec((NC, 400, H), lambda i: (0, i, 0))] * 2,
    out_shape=[jax.ShapeDtypeStruct((NC, N, H), jnp.float32)] * 2,
)


@functools.partial(
    pl.kernel,
    mesh=_mesh,
    out_type=[jax.ShapeDtypeStruct((NC, N, H), jnp.float32),
              jax.ShapeDtypeStruct((NC, N, H), jnp.float32)],
    scratch_types=[
        pltpu.VMEM((C_NB, C_B), jnp.int32),
        pltpu.VMEM((C_NB, C_B), jnp.int32),
        pltpu.VMEM((C_NB, C_B), jnp.int32),
        pltpu.VMEM((C_NB, C_B), jnp.int32),
        pltpu.VMEM((2, C_B, H), jnp.float32),
        pltpu.VMEM_SHARED((ACC_ROWS, H), jnp.float32),
        pltpu.SemaphoreType.DMA,
        pltpu.SemaphoreType.DMA,
    ],
)
def _prop_kernel(srcP, dstP, z1S, z2S, s1_out, s2_out,
                 src_v, dst_v, rem0, rem1, gbuf, acc, sem0, sem1):
    c = lax.axis_index("c")
    t = lax.axis_index("s")
    pltpu.sync_copy(srcP.at[c, t], src_v)
    pltpu.sync_copy(dstP.at[t], dst_v)
    sems = (sem0, sem1)

    def rm_body(i, carry):
        for k in range(C_B // 16):
            d = dst_v[i, pl.ds(16 * k, 16)]
            rem0[i, pl.ds(16 * k, 16)] = jnp.where(
                d < PASS_ROWS, d, PASS_ROWS)
            rem1[i, pl.ds(16 * k, 16)] = jnp.where(
                d >= PASS_ROWS, d - PASS_ROWS, PASS_ROWS)
        return carry

    lax.fori_loop(0, C_NB, rm_body, 0)

    def sweep(zS, s_out, base, rem):
        def g_start(j, b):
            pltpu.make_async_copy(zS.at[src_v.at[j]], gbuf.at[b],
                                  sems[b]).start()

        def g_wait(j, b):
            pltpu.make_async_copy(zS.at[src_v.at[j]], gbuf.at[b],
                                  sems[b]).wait()

        g_start(0, 0)
        g_start(1, 1)
        pltpu.sync_copy(zS.at[pl.ds(c * N + base + t * RPT, RPT)],
                        acc.at[pl.ds(t * RPT, RPT)])

        @pl.when(t == 0)
        def _():
            pltpu.sync_copy(zS.at[pl.ds(c * N + base + TAIL_R0, TAIL)],
                            acc.at[pl.ds(TAIL_R0, TAIL)])

        plsc.subcore_barrier()

        def body(i, carry):
            j0 = 2 * i
            for b in range(2):
                j = j0 + b

                @pl.when(j < C_NB)
                def _():
                    g_wait(j, b)
                    pltpu.sync_copy(gbuf.at[b], acc.at[rem.at[j]], add=True)

                    @pl.when(j + 2 < C_NB)
                    def _():
                        g_start(j + 2, b)

            return carry

        lax.fori_loop(0, (C_NB + 1) // 2, body, 0)
        plsc.subcore_barrier()
        pltpu.sync_copy(acc.at[pl.ds(t * RPT, RPT)],
                        s_out.at[c, pl.ds(base + t * RPT, RPT)])

        @pl.when(t == 0)
        def _():
            pltpu.sync_copy(acc.at[pl.ds(TAIL_R0, TAIL)],
                            s_out.at[c, pl.ds(base + TAIL_R0, TAIL)])

    sweep(z1S, s1_out, 0, rem0)
    sweep(z1S, s1_out, PASS_ROWS, rem1)
    sweep(z2S, s2_out, 0, rem0)
    sweep(z2S, s2_out, PASS_ROWS, rem1)


def _out_body(s1_ref, s2_ref, deg_ref, h_ref, x_ref):
    dinv = lax.rsqrt(deg_ref[...])
    s1 = s1_ref[...]
    s2 = s2_ref[...]
    x_ref[...] = jnp.concatenate([s1[0], s1[1]], axis=1) * dinv
    h_ref[...] = jnp.concatenate([s2[0], s2[1]], axis=1) * dinv


_out = pl.pallas_call(
    _out_body,
    grid=(25,),
    in_specs=[
        pl.BlockSpec((NC, 400, H), lambda i: (0, i, 0)),
        pl.BlockSpec((NC, 400, H), lambda i: (0, i, 0)),
        pl.BlockSpec((400, 1), lambda i: (i, 0)),
    ],
    out_specs=[pl.BlockSpec((400, D), lambda i: (i, 0))] * 2,
    out_shape=[jax.ShapeDtypeStruct((N, D), jnp.float32)] * 2,
)


def kernel(x, edge_index, W1, b1, W2, b2):
    src = edge_index[0]
    dst = edge_index[1]
    padA = jnp.full((A_TOTAL - E,), JUNK, jnp.int32)
    dstA = jnp.concatenate([dst, padA]).reshape(NC * NS, A_BATCHES, 128)
    c0, c1 = _deg_kernel(dstA)
    deg_col = (c0 + c1 + 1.0).reshape(N, 1)
    zp1, zp2 = _lin(x, W1, b1.reshape(1, D), W2, b2.reshape(1, D), deg_col)
    srcG = src.reshape(NS, C_NB, C_B)
    srcP = jnp.stack([srcG, srcG + N])
    dstP = dst.reshape(NS, C_NB, C_B)
    s1, s2 = _prop_kernel(srcP, dstP, zp1.reshape(NC * N, H),
                          zp2.reshape(NC * N, H))
    h, x_ = _out(s1, s2, deg_col)
    return (h, x_)
